# baseline (speedup 1.0000x reference)
import functools

import jax
import jax.numpy as jnp
from jax import lax
from jax.experimental import pallas as pl
from jax.experimental.pallas import tpu as pltpu
from jax.experimental.pallas import tpu_sc as plsc

_NC = 2
_NS = 16
_NW = _NC * _NS

_BM = 1024


def _router_body(x_ref, wr_ref, br_ref, wts_ref, dst_ref, cnt_ref):
    n = x_ref.shape[0]
    e = wr_ref.shape[1]
    logits = jnp.dot(x_ref[...], wr_ref[...],
                     preferred_element_type=jnp.float32) + br_ref[...]
    m = jnp.max(logits, axis=-1, keepdims=True)
    s = jnp.sum(jnp.exp(logits - m), axis=-1)
    wts_ref[...] = 1.0 / s
    eid = jnp.argmax(logits, axis=-1).astype(jnp.int32)

    oh = (eid[:, None] == lax.broadcasted_iota(jnp.int32, (1, e), 1))
    ohi = oh.astype(jnp.int32)
    csum = ohi
    k = 1
    while k < n:
        shifted = jnp.concatenate(
            [jnp.zeros((k, e), jnp.int32), csum[:-k, :]], axis=0)
        csum = csum + shifted
        k *= 2
    counts = csum[n - 1:n, :]
    off = counts
    k = 1
    while k < e:
        off_sh = jnp.concatenate(
            [jnp.zeros((1, k), jnp.int32), off[:, :-k]], axis=1)
        off = off + off_sh
        k *= 2
    off_ex = off - counts
    rank = jnp.sum(ohi * csum, axis=1) - 1
    base = jnp.sum(ohi * off_ex, axis=1)
    dst_ref[...] = base + rank
    cnt_ref[...] = counts[0]


def _router_dispatch(x2, Wr, br):
    n, d = x2.shape
    e = Wr.shape[1]
    return pl.pallas_call(
        _router_body,
        out_shape=(jax.ShapeDtypeStruct((n,), jnp.float32),
                   jax.ShapeDtypeStruct((n,), jnp.int32),
                   jax.ShapeDtypeStruct((e,), jnp.int32)),
    )(x2, Wr, br.reshape(1, e))


def _step_tables(counts, n, e):
    nblk = n // _BM
    g_tot = nblk + e - 1
    off = jnp.concatenate(
        [jnp.zeros((1,), jnp.int32), jnp.cumsum(counts).astype(jnp.int32)])
    first_blk = off[:e] // _BM
    last_blk = jnp.maximum(off[1:] - 1, 0) // _BM
    pairs = jnp.where(counts > 0, last_blk - first_blk + 1, 0).astype(jnp.int32)
    cum = jnp.cumsum(pairs)
    start = cum - pairs
    s = jnp.arange(g_tot, dtype=jnp.int32)
    valid = s < cum[-1]
    e_s = jnp.clip(jnp.searchsorted(cum, s, side="right"), 0, e - 1)
    e_s = e_s.astype(jnp.int32)
    m_s = first_blk[e_s] + (s - start[e_s])
    m_s = jnp.where(valid, m_s, nblk - 1).astype(jnp.int32)
    lo_s = jnp.where(valid, jnp.maximum(off[e_s], m_s * _BM), 0).astype(jnp.int32)
    hi_s = jnp.where(valid, jnp.minimum(off[e_s + 1], (m_s + 1) * _BM),
                     0).astype(jnp.int32)
    e_last = jnp.clip(jnp.searchsorted(cum, cum[-1] - 1, side="right"), 0, e - 1)
    e_s = jnp.where(valid, e_s, e_last).astype(jnp.int32)
    return m_s, e_s, lo_s, hi_s


def _moe_body(mb_ref, eb_ref, lo_ref, hi_ref,
              xs_ref, w_ref, b_ref, ws_ref, out_ref):
    g = pl.program_id(0)
    m = mb_ref[g]
    lo = lo_ref[g]
    hi = hi_ref[g]
    t = jnp.dot(xs_ref[...], w_ref[0], preferred_element_type=jnp.float32)
    t = (t + b_ref[0]) * ws_ref[0][0][:, None]
    r = m * _BM + lax.broadcasted_iota(jnp.int32, (_BM, 1), 0)
    t = jnp.where((r >= lo) & (r < hi), t, 0.0)
    first = (g == 0) | (m != mb_ref[jnp.maximum(g - 1, 0)])

    @pl.when(first)
    def _():
        out_ref[...] = t

    @pl.when(jnp.logical_not(first))
    def _():
        out_ref[...] += t


def _moe_grouped(xs, ws, W, b, mblk, eidx, lo, hi):
    n, d = xs.shape
    e = W.shape[0]
    nblk = n // _BM
    g_tot = nblk + e - 1
    ws3 = ws.reshape(nblk, 1, _BM)
    grid_spec = pltpu.PrefetchScalarGridSpec(
        num_scalar_prefetch=4,
        grid=(g_tot,),
        in_specs=[
            pl.BlockSpec((_BM, d), lambda g, mb, eb, lo, hi: (mb[g], 0)),
            pl.BlockSpec((1, d, d), lambda g, mb, eb, lo, hi: (eb[g], 0, 0)),
            pl.BlockSpec((1, 1, d), lambda g, mb, eb, lo, hi: (eb[g], 0, 0)),
            pl.BlockSpec((1, 1, _BM), lambda g, mb, eb, lo, hi: (mb[g], 0, 0)),
        ],
        out_specs=pl.BlockSpec((_BM, d), lambda g, mb, eb, lo, hi: (mb[g], 0)),
    )
    return pl.pallas_call(
        _moe_body,
        grid_spec=grid_spec,
        out_shape=jax.ShapeDtypeStruct((n, d), jnp.float32),
    )(mblk, eidx, lo, hi, xs, W, b.reshape(e, 1, d), ws3)


def _sc_scatter_in(x2, wts, dst):
    n, d = x2.shape
    xdt = x2.dtype
    rpw = n // _NW
    ch = 32
    nch = rpw // ch
    mesh = plsc.VectorSubcoreMesh(core_axis_name="c", subcore_axis_name="s")

    @functools.partial(
        pl.kernel, mesh=mesh,
        out_type=[jax.ShapeDtypeStruct((n, d), xdt),
                  jax.ShapeDtypeStruct((n,), jnp.float32)],
        scratch_types=[
            pltpu.VMEM((nch, ch), jnp.int32),
            pltpu.VMEM((ch, d), xdt),
            pltpu.VMEM((ch, d), xdt),
            pltpu.VMEM((ch, d), xdt),
            pltpu.VMEM((rpw,), jnp.float32),
            pltpu.VMEM((rpw,), jnp.int32),
            pltpu.SemaphoreType.DMA,
            pltpu.SemaphoreType.DMA,
            pltpu.SemaphoreType.DMA,
            pltpu.SemaphoreType.DMA,
            pltpu.SemaphoreType.DMA,
            pltpu.SemaphoreType.DMA,
            pltpu.SemaphoreType.DMA,
        ],
    )
    def k(x_hbm, wts_hbm, dst_hbm, dst2_hbm, xs_hbm, ws_hbm,
          idx_v, buf0, buf1, buf2, wbuf_v, widx_v,
          ls0, ls1, ls2, ss0, ss1, ss2, semw):
        wid = lax.axis_index("s") * _NC + lax.axis_index("c")
        base = wid * rpw
        pltpu.sync_copy(dst2_hbm.at[pl.ds(wid * nch, nch)], idx_v)
        pltpu.sync_copy(dst_hbm.at[pl.ds(base, rpw)], widx_v)
        pltpu.sync_copy(wts_hbm.at[pl.ds(base, rpw)], wbuf_v)
        wcp = pltpu.async_copy(wbuf_v, ws_hbm.at[widx_v], semw)
        bufs = (buf0, buf1, buf2)
        lsems = (ls0, ls1, ls2)
        ssems = (ss0, ss1, ss2)
        ld = [None] * nch
        st = [None] * nch
        for c in range(min(3, nch)):
            ld[c] = pltpu.async_copy(x_hbm.at[pl.ds(base + c * ch, ch)],
                                     bufs[c % 3], lsems[c % 3])
        for c in range(nch):
            ld[c].wait()
            st[c] = pltpu.async_copy(bufs[c % 3], xs_hbm.at[idx_v.at[c]],
                                     ssems[c % 3])
            nxt = c + 3
            if nxt < nch:
                st[c].wait()
                ld[nxt] = pltpu.async_copy(
                    x_hbm.at[pl.ds(base + nxt * ch, ch)],
                    bufs[nxt % 3], lsems[nxt % 3])
        wcp.wait()
        for c in range(max(nch - 3, 0), nch):
            st[c].wait()

    return k(x2, wts, dst, dst.reshape(n // ch, ch))


def _sc_gather_out(ys, dst):
    n, d = ys.shape
    rpw = n // _NW
    ch = 32
    nch = rpw // ch
    mesh = plsc.VectorSubcoreMesh(core_axis_name="c", subcore_axis_name="s")

    @functools.partial(
        pl.kernel, mesh=mesh,
        out_type=jax.ShapeDtypeStruct((n, d), jnp.float32),
        scratch_types=[
            pltpu.VMEM((rpw,), jnp.int32),
            pltpu.VMEM((ch, d), jnp.float32),
            pltpu.VMEM((ch, d), jnp.float32),
            pltpu.VMEM((ch, d), jnp.float32),
            pltpu.SemaphoreType.DMA,
            pltpu.SemaphoreType.DMA,
            pltpu.SemaphoreType.DMA,
            pltpu.SemaphoreType.DMA,
            pltpu.SemaphoreType.DMA,
            pltpu.SemaphoreType.DMA,
        ],
    )
    def k(ys_hbm, dst_hbm, out_hbm, idx_v, buf0, buf1, buf2,
          ls0, ls1, ls2, ss0, ss1, ss2):
        wid = lax.axis_index("s") * _NC + lax.axis_index("c")
        base = wid * rpw
        pltpu.sync_copy(dst_hbm.at[pl.ds(base, rpw)], idx_v)
        bufs = (buf0, buf1, buf2)
        lsems = (ls0, ls1, ls2)
        ssems = (ss0, ss1, ss2)
        ld = [None] * nch
        st = [None] * nch
        for c in range(min(3, nch)):
            ld[c] = pltpu.async_copy(
                ys_hbm.at[idx_v.at[pl.ds(c * ch, ch)]], bufs[c % 3],
                lsems[c % 3])
        for c in range(nch):
            ld[c].wait()
            st[c] = pltpu.async_copy(
                bufs[c % 3], out_hbm.at[pl.ds(base + c * ch, ch)],
                ssems[c % 3])
            nxt = c + 3
            if nxt < nch:
                st[c].wait()
                ld[nxt] = pltpu.async_copy(
                    ys_hbm.at[idx_v.at[pl.ds(nxt * ch, ch)]],
                    bufs[nxt % 3], lsems[nxt % 3])
        for c in range(max(nch - 3, 0), nch):
            st[c].wait()

    return k(ys, dst)


def kernel(x, Wr, br, W, b):
    bsz, t, d = x.shape
    e = Wr.shape[1]
    n = bsz * t
    x2 = x.reshape(n, d)

    wts, dst, counts = _router_dispatch(x2, Wr, br)
    mblk, eidx, lo, hi = _step_tables(counts, n, e)
    xs, ws = _sc_scatter_in(x2, wts, dst)
    ys = _moe_grouped(xs, ws, W, b, mblk, eidx, lo, hi)
    out2 = _sc_gather_out(ys, dst)
    return out2.reshape(bsz, t, d)


# speedup vs baseline: 1.1081x; 1.1081x over previous
import functools

import jax
import jax.numpy as jnp
from jax import lax
from jax.experimental import pallas as pl
from jax.experimental.pallas import tpu as pltpu
from jax.experimental.pallas import tpu_sc as plsc

_NC = 2
_NS = 16
_NW = _NC * _NS

_BM = 512


def _router_body(x_ref, wr_ref, br_ref, wts_ref, dst_ref, cnt_ref):
    n = x_ref.shape[0]
    e = wr_ref.shape[1]
    logits = jnp.dot(x_ref[...], wr_ref[...],
                     preferred_element_type=jnp.float32) + br_ref[...]
    m = jnp.max(logits, axis=-1, keepdims=True)
    s = jnp.sum(jnp.exp(logits - m), axis=-1)
    wts_ref[...] = 1.0 / s
    eid = jnp.argmax(logits, axis=-1).astype(jnp.int32)

    oh = (eid[:, None] == lax.broadcasted_iota(jnp.int32, (1, e), 1))
    ohi = oh.astype(jnp.int32)
    csum = ohi
    k = 1
    while k < n:
        shifted = jnp.concatenate(
            [jnp.zeros((k, e), jnp.int32), csum[:-k, :]], axis=0)
        csum = csum + shifted
        k *= 2
    counts = csum[n - 1:n, :]
    off = counts
    k = 1
    while k < e:
        off_sh = jnp.concatenate(
            [jnp.zeros((1, k), jnp.int32), off[:, :-k]], axis=1)
        off = off + off_sh
        k *= 2
    off_ex = off - counts
    rank = jnp.sum(ohi * csum, axis=1) - 1
    base = jnp.sum(ohi * off_ex, axis=1)
    dst_ref[...] = base + rank
    cnt_ref[...] = counts[0]


def _router_dispatch(x2, Wr, br):
    n, d = x2.shape
    e = Wr.shape[1]
    return pl.pallas_call(
        _router_body,
        out_shape=(jax.ShapeDtypeStruct((n,), jnp.float32),
                   jax.ShapeDtypeStruct((n,), jnp.int32),
                   jax.ShapeDtypeStruct((e,), jnp.int32)),
    )(x2, Wr, br.reshape(1, e))


def _step_tables(counts, n, e):
    nblk = n // _BM
    g_tot = nblk + e - 1
    off = jnp.concatenate(
        [jnp.zeros((1,), jnp.int32), jnp.cumsum(counts).astype(jnp.int32)])
    first_blk = off[:e] // _BM
    last_blk = jnp.maximum(off[1:] - 1, 0) // _BM
    pairs = jnp.where(counts > 0, last_blk - first_blk + 1, 0).astype(jnp.int32)
    cum = jnp.cumsum(pairs)
    start = cum - pairs
    s = jnp.arange(g_tot, dtype=jnp.int32)
    valid = s < cum[-1]
    e_s = jnp.clip(jnp.searchsorted(cum, s, side="right"), 0, e - 1)
    e_s = e_s.astype(jnp.int32)
    m_s = first_blk[e_s] + (s - start[e_s])
    m_s = jnp.where(valid, m_s, nblk - 1).astype(jnp.int32)
    lo_s = jnp.where(valid, jnp.maximum(off[e_s], m_s * _BM), 0).astype(jnp.int32)
    hi_s = jnp.where(valid, jnp.minimum(off[e_s + 1], (m_s + 1) * _BM),
                     0).astype(jnp.int32)
    e_last = jnp.clip(jnp.searchsorted(cum, cum[-1] - 1, side="right"), 0, e - 1)
    e_s = jnp.where(valid, e_s, e_last).astype(jnp.int32)
    return m_s, e_s, lo_s, hi_s


def _moe_body(mb_ref, eb_ref, lo_ref, hi_ref,
              xs_ref, w_ref, b_ref, ws_ref, out_ref):
    g = pl.program_id(0)
    m = mb_ref[g]
    lo = lo_ref[g]
    hi = hi_ref[g]
    t = jnp.dot(xs_ref[...], w_ref[0], preferred_element_type=jnp.float32)
    t = (t + b_ref[0]) * ws_ref[0][0][:, None]
    r = m * _BM + lax.broadcasted_iota(jnp.int32, (_BM, 1), 0)
    t = jnp.where((r >= lo) & (r < hi), t, 0.0)
    first = (g == 0) | (m != mb_ref[jnp.maximum(g - 1, 0)])

    @pl.when(first)
    def _():
        out_ref[...] = t

    @pl.when(jnp.logical_not(first))
    def _():
        out_ref[...] += t


def _moe_grouped(xs, ws, W, b, mblk, eidx, lo, hi):
    n, d = xs.shape
    e = W.shape[0]
    nblk = n // _BM
    g_tot = nblk + e - 1
    ws3 = ws.reshape(nblk, 1, _BM)
    grid_spec = pltpu.PrefetchScalarGridSpec(
        num_scalar_prefetch=4,
        grid=(g_tot,),
        in_specs=[
            pl.BlockSpec((_BM, d), lambda g, mb, eb, lo, hi: (mb[g], 0)),
            pl.BlockSpec((1, d, d), lambda g, mb, eb, lo, hi: (eb[g], 0, 0)),
            pl.BlockSpec((1, 1, d), lambda g, mb, eb, lo, hi: (eb[g], 0, 0)),
            pl.BlockSpec((1, 1, _BM), lambda g, mb, eb, lo, hi: (mb[g], 0, 0)),
        ],
        out_specs=pl.BlockSpec((_BM, d), lambda g, mb, eb, lo, hi: (mb[g], 0)),
    )
    return pl.pallas_call(
        _moe_body,
        grid_spec=grid_spec,
        out_shape=jax.ShapeDtypeStruct((n, d), jnp.float32),
    )(mblk, eidx, lo, hi, xs, W, b.reshape(e, 1, d), ws3)


def _sc_scatter_in(x2, wts, dst):
    n, d = x2.shape
    xdt = x2.dtype
    rpw = n // _NW
    ch = 32
    nch = rpw // ch
    mesh = plsc.VectorSubcoreMesh(core_axis_name="c", subcore_axis_name="s")

    @functools.partial(
        pl.kernel, mesh=mesh,
        out_type=[jax.ShapeDtypeStruct((n, d), xdt),
                  jax.ShapeDtypeStruct((n,), jnp.float32)],
        scratch_types=[
            pltpu.VMEM((nch, ch), jnp.int32),
            pltpu.VMEM((ch, d), xdt),
            pltpu.VMEM((ch, d), xdt),
            pltpu.VMEM((ch, d), xdt),
            pltpu.VMEM((rpw,), jnp.float32),
            pltpu.VMEM((rpw,), jnp.int32),
            pltpu.SemaphoreType.DMA,
            pltpu.SemaphoreType.DMA,
            pltpu.SemaphoreType.DMA,
            pltpu.SemaphoreType.DMA,
            pltpu.SemaphoreType.DMA,
            pltpu.SemaphoreType.DMA,
            pltpu.SemaphoreType.DMA,
        ],
    )
    def k(x_hbm, wts_hbm, dst_hbm, dst2_hbm, xs_hbm, ws_hbm,
          idx_v, buf0, buf1, buf2, wbuf_v, widx_v,
          ls0, ls1, ls2, ss0, ss1, ss2, semw):
        wid = lax.axis_index("s") * _NC + lax.axis_index("c")
        base = wid * rpw
        pltpu.sync_copy(dst2_hbm.at[pl.ds(wid * nch, nch)], idx_v)
        pltpu.sync_copy(dst_hbm.at[pl.ds(base, rpw)], widx_v)
        pltpu.sync_copy(wts_hbm.at[pl.ds(base, rpw)], wbuf_v)
        wcp = pltpu.async_copy(wbuf_v, ws_hbm.at[widx_v], semw)
        bufs = (buf0, buf1, buf2)
        lsems = (ls0, ls1, ls2)
        ssems = (ss0, ss1, ss2)
        ld = [None] * nch
        st = [None] * nch
        for c in range(min(3, nch)):
            ld[c] = pltpu.async_copy(x_hbm.at[pl.ds(base + c * ch, ch)],
                                     bufs[c % 3], lsems[c % 3])
        for c in range(nch):
            ld[c].wait()
            st[c] = pltpu.async_copy(bufs[c % 3], xs_hbm.at[idx_v.at[c]],
                                     ssems[c % 3])
            nxt = c + 3
            if nxt < nch:
                st[c].wait()
                ld[nxt] = pltpu.async_copy(
                    x_hbm.at[pl.ds(base + nxt * ch, ch)],
                    bufs[nxt % 3], lsems[nxt % 3])
        wcp.wait()
        for c in range(max(nch - 3, 0), nch):
            st[c].wait()

    return k(x2, wts, dst, dst.reshape(n // ch, ch))


def _sc_gather_out(ys, dst):
    n, d = ys.shape
    rpw = n // _NW
    ch = 32
    nch = rpw // ch
    mesh = plsc.VectorSubcoreMesh(core_axis_name="c", subcore_axis_name="s")

    @functools.partial(
        pl.kernel, mesh=mesh,
        out_type=jax.ShapeDtypeStruct((n, d), jnp.float32),
        scratch_types=[
            pltpu.VMEM((rpw,), jnp.int32),
            pltpu.VMEM((ch, d), jnp.float32),
            pltpu.VMEM((ch, d), jnp.float32),
            pltpu.VMEM((ch, d), jnp.float32),
            pltpu.SemaphoreType.DMA,
            pltpu.SemaphoreType.DMA,
            pltpu.SemaphoreType.DMA,
            pltpu.SemaphoreType.DMA,
            pltpu.SemaphoreType.DMA,
            pltpu.SemaphoreType.DMA,
        ],
    )
    def k(ys_hbm, dst_hbm, out_hbm, idx_v, buf0, buf1, buf2,
          ls0, ls1, ls2, ss0, ss1, ss2):
        wid = lax.axis_index("s") * _NC + lax.axis_index("c")
        base = wid * rpw
        pltpu.sync_copy(dst_hbm.at[pl.ds(base, rpw)], idx_v)
        bufs = (buf0, buf1, buf2)
        lsems = (ls0, ls1, ls2)
        ssems = (ss0, ss1, ss2)
        ld = [None] * nch
        st = [None] * nch
        for c in range(min(3, nch)):
            ld[c] = pltpu.async_copy(
                ys_hbm.at[idx_v.at[pl.ds(c * ch, ch)]], bufs[c % 3],
                lsems[c % 3])
        for c in range(nch):
            ld[c].wait()
            st[c] = pltpu.async_copy(
                bufs[c % 3], out_hbm.at[pl.ds(base + c * ch, ch)],
                ssems[c % 3])
            nxt = c + 3
            if nxt < nch:
                st[c].wait()
                ld[nxt] = pltpu.async_copy(
                    ys_hbm.at[idx_v.at[pl.ds(nxt * ch, ch)]],
                    bufs[nxt % 3], lsems[nxt % 3])
        for c in range(max(nch - 3, 0), nch):
            st[c].wait()

    return k(ys, dst)


def kernel(x, Wr, br, W, b):
    bsz, t, d = x.shape
    e = Wr.shape[1]
    n = bsz * t
    x2 = x.reshape(n, d)

    wts, dst, counts = _router_dispatch(x2, Wr, br)
    mblk, eidx, lo, hi = _step_tables(counts, n, e)
    xs, ws = _sc_scatter_in(x2, wts, dst)
    ys = _moe_grouped(xs, ws, W, b, mblk, eidx, lo, hi)
    out2 = _sc_gather_out(ys, dst)
    return out2.reshape(bsz, t, d)
